# baseline (speedup 1.0000x reference)
import functools

import jax
import jax.numpy as jnp
from jax import lax
from jax.experimental import pallas as pl
from jax.experimental.pallas import tpu as pltpu
from jax.experimental.pallas import tpu_sc as plsc

E = 64
D = 768
TB = 1024
NW = 32
C = 256
L = 16
T = 32768
NSEG = 4
TS = T // NSEG
TPW = TS // NW


def _tc_logits_body(x_ref, w_ref, b_ref, lg_ref):
    xb = x_ref[...]
    w = w_ref[...]
    lg_ref[...] = jax.lax.dot_general(
        w, xb, (((1,), (1,)), ((), ())),
        preferred_element_type=jnp.float32) + b_ref[...]


def _sc_route_body(seg, lg_hbm, rw_hbm, idx_hbm,
                   lv0, lv1, rv0, rv1, iv0, iv1,
                   si0, si1, sr0, sr1, sx0, sx1):
    wid = lax.axis_index("s") * 2 + lax.axis_index("c")
    lane = lax.iota(jnp.int32, L)
    neg_inf = jnp.full((L,), -jnp.inf, dtype=jnp.float32)
    zero_i = jnp.zeros((L,), dtype=jnp.int32)
    zeros_f = jnp.zeros((L,), dtype=jnp.float32)
    lvs, rvs, ivs = (lv0, lv1), (rv0, rv1), (iv0, iv1)
    sins, srws, sxs = (si0, si1), (sr0, sr1), (sx0, sx1)

    NCH = TPW // C
    base0 = wid * TPW

    def start_in(ch):
        return pltpu.async_copy(
            lg_hbm.at[:, pl.ds(base0 + ch * C, C)], lvs[ch % 2],
            sins[ch % 2])

    in_h = [start_in(0)]
    out_h = [None, None]

    for ch in range(NCH):
        tbase = seg * TS + base0 + ch * C
        sl = ch % 2
        lv, rv, iv = lvs[sl], rvs[sl], ivs[sl]
        if ch + 1 < NCH:
            in_h.append(start_in(ch + 1))
        in_h[ch].wait()
        if out_h[sl] is not None:
            for h in out_h[sl]:
                h.wait()

        def zbody(j, _):
            for r in range(E):
                rv[r, pl.ds(j * L, L)] = zeros_f
            return 0
        lax.fori_loop(0, C // L, zbody, 0)

        def gbody(g, _):
            goff = g * L
            m1, i1 = neg_inf, zero_i
            m2, i2 = neg_inf, zero_i
            for e in range(E):
                v = lv[e, pl.ds(goff, L)]
                es = jnp.full((L,), e, dtype=jnp.int32)
                gt1 = v > m1
                gt2 = v > m2
                nm2 = jnp.where(gt1, m1, jnp.where(gt2, v, m2))
                ni2 = jnp.where(gt1, i1, jnp.where(gt2, es, i2))
                m1 = jnp.where(gt1, v, m1)
                i1 = jnp.where(gt1, es, i1)
                m2, i2 = nm2, ni2
            ew = jnp.exp(m2 - m1)
            s = 1.0 / (1.0 + ew)
            w1 = s
            w2 = ew * s
            tok = goff + lane
            plsc.store_scatter(rv, [i1, tok], w1)
            plsc.store_scatter(rv, [i2, tok], w2)
            offb = (goff // 128) * 256 + goff % 128
            plsc.store_scatter(iv, [offb + lane], i1)
            plsc.store_scatter(iv, [offb + 128 + lane], i2)
            return 0
        lax.fori_loop(0, C // L, gbody, 0)

        out_h[sl] = (
            pltpu.async_copy(rv, rw_hbm.at[:, pl.ds(tbase, C)], srws[sl]),
            pltpu.async_copy(iv, idx_hbm.at[pl.ds(tbase * 2, C * 2)],
                             sxs[sl]),
        )

    for hs in out_h:
        for h in hs or ():
            h.wait()


_SC_SCRATCH = [
    pltpu.VMEM((E, C), jnp.float32),
    pltpu.VMEM((E, C), jnp.float32),
    pltpu.VMEM((E, C), jnp.float32),
    pltpu.VMEM((E, C), jnp.float32),
    pltpu.VMEM((C * 2,), jnp.int32),
    pltpu.VMEM((C * 2,), jnp.int32),
    pltpu.SemaphoreType.DMA,
    pltpu.SemaphoreType.DMA,
    pltpu.SemaphoreType.DMA,
    pltpu.SemaphoreType.DMA,
    pltpu.SemaphoreType.DMA,
    pltpu.SemaphoreType.DMA,
]


def kernel(x, W, b):
    xf = x.reshape(T, D)
    b2 = b.reshape(E, 1)
    mesh = plsc.VectorSubcoreMesh(core_axis_name="c", subcore_axis_name="s")

    def tc_logits(seg):
        off = seg * (TS // TB)
        return pl.pallas_call(
            _tc_logits_body,
            grid=(TS // TB,),
            in_specs=[
                pl.BlockSpec((TB, D), lambda i: (i + off, 0)),
                pl.BlockSpec((E, D), lambda i: (0, 0)),
                pl.BlockSpec((E, 1), lambda i: (0, 0)),
            ],
            out_specs=pl.BlockSpec((E, TB), lambda i: (0, i)),
            out_shape=jax.ShapeDtypeStruct((E, TS), jnp.float32),
        )(xf, W, b2)

    sc0 = functools.partial(
        pl.kernel,
        out_type=[
            jax.ShapeDtypeStruct((E, T), jnp.float32),
            jax.ShapeDtypeStruct((T * 2,), jnp.int32),
        ],
        mesh=mesh,
        compiler_params=pltpu.CompilerParams(needs_layout_passes=False),
        scratch_types=_SC_SCRATCH,
    )(functools.partial(_sc_route_body, 0))

    sc_refs = [
        functools.partial(
            pl.kernel,
            out_type=(),
            mesh=mesh,
            compiler_params=pltpu.CompilerParams(needs_layout_passes=False),
            scratch_types=_SC_SCRATCH,
        )(functools.partial(_sc_route_body, s))
        for s in range(1, NSEG)
    ]

    lg0 = tc_logits(0)
    rw0, idx0 = sc0(lg0)
    rw_ref = jax.new_ref(rw0)
    idx_ref = jax.new_ref(idx0)
    for s in range(1, NSEG):
        lg = tc_logits(s)
        sc_refs[s - 1](lg, rw_ref, idx_ref)
    rw_t = rw_ref[...]
    idx_flat = idx_ref[...]
    rw = rw_t.T
    idx = idx_flat.reshape(T // 128, 2, 128).transpose(0, 2, 1).reshape(T, 2)
    return (rw, idx)


# speedup vs baseline: 1.0076x; 1.0076x over previous
import functools

import jax
import jax.numpy as jnp
from jax import lax
from jax.experimental import pallas as pl
from jax.experimental.pallas import tpu as pltpu
from jax.experimental.pallas import tpu_sc as plsc

E = 64
D = 768
TB = 1024
NW = 32
C = 256
L = 16
T = 32768
SEGS = ((0, 24576), (24576, 8192))


def _tc_logits_body(x_ref, w_ref, b_ref, lg_ref):
    xb = x_ref[...]
    w = w_ref[...]
    lg_ref[...] = jax.lax.dot_general(
        w, xb, (((1,), (1,)), ((), ())),
        preferred_element_type=jnp.float32) + b_ref[...]


def _sc_route_body(seg_base, seg_len, lg_hbm, rw_hbm, idx_hbm,
                   lv0, lv1, rv0, rv1, iv0, iv1,
                   si0, si1, sr0, sr1, sx0, sx1):
    TPW = seg_len // NW
    wid = lax.axis_index("s") * 2 + lax.axis_index("c")
    lane = lax.iota(jnp.int32, L)
    neg_inf = jnp.full((L,), -jnp.inf, dtype=jnp.float32)
    zero_i = jnp.zeros((L,), dtype=jnp.int32)
    zeros_f = jnp.zeros((L,), dtype=jnp.float32)
    lvs, rvs, ivs = (lv0, lv1), (rv0, rv1), (iv0, iv1)
    sins, srws, sxs = (si0, si1), (sr0, sr1), (sx0, sx1)

    NCH = TPW // C
    base0 = wid * TPW

    def start_in(ch):
        return pltpu.async_copy(
            lg_hbm.at[:, pl.ds(base0 + ch * C, C)], lvs[ch % 2],
            sins[ch % 2])

    in_h = [start_in(0)]
    out_h = [None, None]

    for ch in range(NCH):
        tbase = seg_base + base0 + ch * C
        sl = ch % 2
        lv, rv, iv = lvs[sl], rvs[sl], ivs[sl]
        if ch + 1 < NCH:
            in_h.append(start_in(ch + 1))
        in_h[ch].wait()
        if out_h[sl] is not None:
            for h in out_h[sl]:
                h.wait()

        def zbody(j, _):
            for r in range(E):
                rv[r, pl.ds(j * L, L)] = zeros_f
            return 0
        lax.fori_loop(0, C // L, zbody, 0)

        def gbody(g, _):
            goff = g * L
            m1, i1 = neg_inf, zero_i
            m2, i2 = neg_inf, zero_i
            for e in range(E):
                v = lv[e, pl.ds(goff, L)]
                es = jnp.full((L,), e, dtype=jnp.int32)
                gt1 = v > m1
                gt2 = v > m2
                nm2 = jnp.where(gt1, m1, jnp.where(gt2, v, m2))
                ni2 = jnp.where(gt1, i1, jnp.where(gt2, es, i2))
                m1 = jnp.where(gt1, v, m1)
                i1 = jnp.where(gt1, es, i1)
                m2, i2 = nm2, ni2
            ew = jnp.exp(m2 - m1)
            s = 1.0 / (1.0 + ew)
            w1 = s
            w2 = ew * s
            tok = goff + lane
            plsc.store_scatter(rv, [i1, tok], w1)
            plsc.store_scatter(rv, [i2, tok], w2)
            offb = (goff // 128) * 256 + goff % 128
            plsc.store_scatter(iv, [offb + lane], i1)
            plsc.store_scatter(iv, [offb + 128 + lane], i2)
            return 0
        lax.fori_loop(0, C // L, gbody, 0)

        out_h[sl] = (
            pltpu.async_copy(rv, rw_hbm.at[:, pl.ds(tbase, C)], srws[sl]),
            pltpu.async_copy(iv, idx_hbm.at[pl.ds(tbase * 2, C * 2)],
                             sxs[sl]),
        )

    for hs in out_h:
        for h in hs or ():
            h.wait()


_SC_SCRATCH = [
    pltpu.VMEM((E, C), jnp.float32),
    pltpu.VMEM((E, C), jnp.float32),
    pltpu.VMEM((E, C), jnp.float32),
    pltpu.VMEM((E, C), jnp.float32),
    pltpu.VMEM((C * 2,), jnp.int32),
    pltpu.VMEM((C * 2,), jnp.int32),
    pltpu.SemaphoreType.DMA,
    pltpu.SemaphoreType.DMA,
    pltpu.SemaphoreType.DMA,
    pltpu.SemaphoreType.DMA,
    pltpu.SemaphoreType.DMA,
    pltpu.SemaphoreType.DMA,
]


def kernel(x, W, b):
    xf = x.reshape(T, D)
    b2 = b.reshape(E, 1)
    mesh = plsc.VectorSubcoreMesh(core_axis_name="c", subcore_axis_name="s")

    def tc_logits(base, length):
        off = base // TB
        return pl.pallas_call(
            _tc_logits_body,
            grid=(length // TB,),
            in_specs=[
                pl.BlockSpec((TB, D), lambda i: (i + off, 0)),
                pl.BlockSpec((E, D), lambda i: (0, 0)),
                pl.BlockSpec((E, 1), lambda i: (0, 0)),
            ],
            out_specs=pl.BlockSpec((E, TB), lambda i: (0, i)),
            out_shape=jax.ShapeDtypeStruct((E, length), jnp.float32),
        )(xf, W, b2)

    sc0 = functools.partial(
        pl.kernel,
        out_type=[
            jax.ShapeDtypeStruct((E, T), jnp.float32),
            jax.ShapeDtypeStruct((T * 2,), jnp.int32),
        ],
        mesh=mesh,
        compiler_params=pltpu.CompilerParams(needs_layout_passes=False),
        scratch_types=_SC_SCRATCH,
    )(functools.partial(_sc_route_body, *SEGS[0]))

    sc_refs = [
        functools.partial(
            pl.kernel,
            out_type=(),
            mesh=mesh,
            compiler_params=pltpu.CompilerParams(needs_layout_passes=False),
            scratch_types=_SC_SCRATCH,
        )(functools.partial(_sc_route_body, *SEGS[s]))
        for s in range(1, len(SEGS))
    ]

    lg0 = tc_logits(*SEGS[0])
    rw0, idx0 = sc0(lg0)
    rw_ref = jax.new_ref(rw0)
    idx_ref = jax.new_ref(idx0)
    for s in range(1, len(SEGS)):
        lg = tc_logits(*SEGS[s])
        sc_refs[s - 1](lg, rw_ref, idx_ref)
    rw_t = rw_ref[...]
    idx_flat = idx_ref[...]
    rw = rw_t.T
    idx = idx_flat.reshape(T // 128, 2, 128).transpose(0, 2, 1).reshape(T, 2)
    return (rw, idx)


# speedup vs baseline: 1.0189x; 1.0112x over previous
import functools

import jax
import jax.numpy as jnp
from jax import lax
from jax.experimental import pallas as pl
from jax.experimental.pallas import tpu as pltpu
from jax.experimental.pallas import tpu_sc as plsc

E = 64
D = 768
TB = 1024
NW = 32
C = 256
L = 16
T = 32768
SEGS = ((0, 16384), (16384, 16384))


def _tc_logits_body(x_ref, w_ref, b_ref, lg_ref):
    xb = x_ref[...]
    w = w_ref[...]
    lg_ref[...] = jax.lax.dot_general(
        w, xb, (((1,), (1,)), ((), ())),
        preferred_element_type=jnp.float32) + b_ref[...]


def _sc_route_body(seg_base, seg_len, lg_hbm, rw_hbm, idx_hbm,
                   lv0, lv1, rv0, rv1, iv0, iv1,
                   si0, si1, sr0, sr1, sx0, sx1):
    TPW = seg_len // NW
    wid = lax.axis_index("s") * 2 + lax.axis_index("c")
    lane = lax.iota(jnp.int32, L)
    neg_inf = jnp.full((L,), -jnp.inf, dtype=jnp.float32)
    zero_i = jnp.zeros((L,), dtype=jnp.int32)
    zeros_f = jnp.zeros((L,), dtype=jnp.float32)
    lvs, rvs, ivs = (lv0, lv1), (rv0, rv1), (iv0, iv1)
    sins, srws, sxs = (si0, si1), (sr0, sr1), (sx0, sx1)

    NCH = TPW // C
    base0 = wid * TPW

    def start_in(ch):
        return pltpu.async_copy(
            lg_hbm.at[:, pl.ds(base0 + ch * C, C)], lvs[ch % 2],
            sins[ch % 2])

    in_h = [start_in(0)]
    out_h = [None, None]

    for ch in range(NCH):
        tbase = seg_base + base0 + ch * C
        sl = ch % 2
        lv, rv, iv = lvs[sl], rvs[sl], ivs[sl]
        if ch + 1 < NCH:
            in_h.append(start_in(ch + 1))
        in_h[ch].wait()
        if out_h[sl] is not None:
            for h in out_h[sl]:
                h.wait()

        def zbody(j, _):
            for r in range(E):
                rv[r, pl.ds(j * L, L)] = zeros_f
            return 0
        lax.fori_loop(0, C // L, zbody, 0)

        def gbody(g, _):
            goff = g * L
            m1, i1 = neg_inf, zero_i
            m2, i2 = neg_inf, zero_i
            for e in range(E):
                v = lv[e, pl.ds(goff, L)]
                es = jnp.full((L,), e, dtype=jnp.int32)
                gt1 = v > m1
                gt2 = v > m2
                nm2 = jnp.where(gt1, m1, jnp.where(gt2, v, m2))
                ni2 = jnp.where(gt1, i1, jnp.where(gt2, es, i2))
                m1 = jnp.where(gt1, v, m1)
                i1 = jnp.where(gt1, es, i1)
                m2, i2 = nm2, ni2
            ew = jnp.exp(m2 - m1)
            s = 1.0 / (1.0 + ew)
            w1 = s
            w2 = ew * s
            tok = goff + lane
            plsc.store_scatter(rv, [i1, tok], w1)
            plsc.store_scatter(rv, [i2, tok], w2)
            offb = (goff // 128) * 256 + goff % 128
            plsc.store_scatter(iv, [offb + lane], i1)
            plsc.store_scatter(iv, [offb + 128 + lane], i2)
            return 0
        lax.fori_loop(0, C // L, gbody, 0)

        out_h[sl] = (
            pltpu.async_copy(rv, rw_hbm.at[:, pl.ds(tbase, C)], srws[sl]),
            pltpu.async_copy(iv, idx_hbm.at[pl.ds(tbase * 2, C * 2)],
                             sxs[sl]),
        )

    for hs in out_h:
        for h in hs or ():
            h.wait()


_SC_SCRATCH = [
    pltpu.VMEM((E, C), jnp.float32),
    pltpu.VMEM((E, C), jnp.float32),
    pltpu.VMEM((E, C), jnp.float32),
    pltpu.VMEM((E, C), jnp.float32),
    pltpu.VMEM((C * 2,), jnp.int32),
    pltpu.VMEM((C * 2,), jnp.int32),
    pltpu.SemaphoreType.DMA,
    pltpu.SemaphoreType.DMA,
    pltpu.SemaphoreType.DMA,
    pltpu.SemaphoreType.DMA,
    pltpu.SemaphoreType.DMA,
    pltpu.SemaphoreType.DMA,
]


def kernel(x, W, b):
    xf = x.reshape(T, D)
    b2 = b.reshape(E, 1)
    mesh = plsc.VectorSubcoreMesh(core_axis_name="c", subcore_axis_name="s")

    def tc_logits(base, length):
        off = base // TB
        return pl.pallas_call(
            _tc_logits_body,
            grid=(length // TB,),
            in_specs=[
                pl.BlockSpec((TB, D), lambda i: (i + off, 0)),
                pl.BlockSpec((E, D), lambda i: (0, 0)),
                pl.BlockSpec((E, 1), lambda i: (0, 0)),
            ],
            out_specs=pl.BlockSpec((E, TB), lambda i: (0, i)),
            out_shape=jax.ShapeDtypeStruct((E, length), jnp.float32),
        )(xf, W, b2)

    sc0 = functools.partial(
        pl.kernel,
        out_type=[
            jax.ShapeDtypeStruct((E, T), jnp.float32),
            jax.ShapeDtypeStruct((T * 2,), jnp.int32),
        ],
        mesh=mesh,
        compiler_params=pltpu.CompilerParams(needs_layout_passes=False),
        scratch_types=_SC_SCRATCH,
    )(functools.partial(_sc_route_body, *SEGS[0]))

    sc_refs = [
        functools.partial(
            pl.kernel,
            out_type=(),
            mesh=mesh,
            compiler_params=pltpu.CompilerParams(needs_layout_passes=False),
            scratch_types=_SC_SCRATCH,
        )(functools.partial(_sc_route_body, *SEGS[s]))
        for s in range(1, len(SEGS))
    ]

    lg0 = tc_logits(*SEGS[0])
    rw0, idx0 = sc0(lg0)
    rw_ref = jax.new_ref(rw0)
    idx_ref = jax.new_ref(idx0)
    for s in range(1, len(SEGS)):
        lg = tc_logits(*SEGS[s])
        sc_refs[s - 1](lg, rw_ref, idx_ref)
    rw_t = rw_ref[...]
    idx_flat = idx_ref[...]
    rw = rw_t.T
    idx = idx_flat.reshape(T // 128, 2, 128).transpose(0, 2, 1).reshape(T, 2)
    return (rw, idx)


# speedup vs baseline: 1.0303x; 1.0112x over previous
import functools

import jax
import jax.numpy as jnp
from jax import lax
from jax.experimental import pallas as pl
from jax.experimental.pallas import tpu as pltpu
from jax.experimental.pallas import tpu_sc as plsc

E = 64
D = 768
TB = 1024
NW = 32
C = 256
L = 16
T = 32768
SEGS = ((0, 16384), (16384, 8192), (24576, 8192))


def _tc_logits_body(x_ref, w_ref, b_ref, lg_ref):
    xb = x_ref[...]
    w = w_ref[...]
    lg_ref[...] = jax.lax.dot_general(
        w, xb, (((1,), (1,)), ((), ())),
        preferred_element_type=jnp.float32) + b_ref[...]


def _sc_route_body(seg_base, seg_len, lg_hbm, rw_hbm, idx_hbm,
                   lv0, lv1, rv0, rv1, iv0, iv1,
                   si0, si1, sr0, sr1, sx0, sx1):
    TPW = seg_len // NW
    wid = lax.axis_index("s") * 2 + lax.axis_index("c")
    lane = lax.iota(jnp.int32, L)
    neg_inf = jnp.full((L,), -jnp.inf, dtype=jnp.float32)
    zero_i = jnp.zeros((L,), dtype=jnp.int32)
    zeros_f = jnp.zeros((L,), dtype=jnp.float32)
    lvs, rvs, ivs = (lv0, lv1), (rv0, rv1), (iv0, iv1)
    sins, srws, sxs = (si0, si1), (sr0, sr1), (sx0, sx1)

    NCH = TPW // C
    base0 = wid * TPW

    def start_in(ch):
        return pltpu.async_copy(
            lg_hbm.at[:, pl.ds(base0 + ch * C, C)], lvs[ch % 2],
            sins[ch % 2])

    in_h = [start_in(0)]
    out_h = [None, None]

    for ch in range(NCH):
        tbase = seg_base + base0 + ch * C
        sl = ch % 2
        lv, rv, iv = lvs[sl], rvs[sl], ivs[sl]
        if ch + 1 < NCH:
            in_h.append(start_in(ch + 1))
        in_h[ch].wait()
        if out_h[sl] is not None:
            for h in out_h[sl]:
                h.wait()

        def zbody(j, _):
            for r in range(E):
                rv[r, pl.ds(j * L, L)] = zeros_f
            return 0
        lax.fori_loop(0, C // L, zbody, 0)

        def gbody(g, _):
            goff = g * L
            m1, i1 = neg_inf, zero_i
            m2, i2 = neg_inf, zero_i
            for e in range(E):
                v = lv[e, pl.ds(goff, L)]
                es = jnp.full((L,), e, dtype=jnp.int32)
                gt1 = v > m1
                gt2 = v > m2
                nm2 = jnp.where(gt1, m1, jnp.where(gt2, v, m2))
                ni2 = jnp.where(gt1, i1, jnp.where(gt2, es, i2))
                m1 = jnp.where(gt1, v, m1)
                i1 = jnp.where(gt1, es, i1)
                m2, i2 = nm2, ni2
            ew = jnp.exp(m2 - m1)
            s = 1.0 / (1.0 + ew)
            w1 = s
            w2 = ew * s
            tok = goff + lane
            plsc.store_scatter(rv, [i1, tok], w1)
            plsc.store_scatter(rv, [i2, tok], w2)
            offb = (goff // 128) * 256 + goff % 128
            plsc.store_scatter(iv, [offb + lane], i1)
            plsc.store_scatter(iv, [offb + 128 + lane], i2)
            return 0
        lax.fori_loop(0, C // L, gbody, 0)

        out_h[sl] = (
            pltpu.async_copy(rv, rw_hbm.at[:, pl.ds(tbase, C)], srws[sl]),
            pltpu.async_copy(iv, idx_hbm.at[pl.ds(tbase * 2, C * 2)],
                             sxs[sl]),
        )

    for hs in out_h:
        for h in hs or ():
            h.wait()


_SC_SCRATCH = [
    pltpu.VMEM((E, C), jnp.float32),
    pltpu.VMEM((E, C), jnp.float32),
    pltpu.VMEM((E, C), jnp.float32),
    pltpu.VMEM((E, C), jnp.float32),
    pltpu.VMEM((C * 2,), jnp.int32),
    pltpu.VMEM((C * 2,), jnp.int32),
    pltpu.SemaphoreType.DMA,
    pltpu.SemaphoreType.DMA,
    pltpu.SemaphoreType.DMA,
    pltpu.SemaphoreType.DMA,
    pltpu.SemaphoreType.DMA,
    pltpu.SemaphoreType.DMA,
]


def kernel(x, W, b):
    xf = x.reshape(T, D)
    b2 = b.reshape(E, 1)
    mesh = plsc.VectorSubcoreMesh(core_axis_name="c", subcore_axis_name="s")

    def tc_logits(base, length):
        off = base // TB
        return pl.pallas_call(
            _tc_logits_body,
            grid=(length // TB,),
            in_specs=[
                pl.BlockSpec((TB, D), lambda i: (i + off, 0)),
                pl.BlockSpec((E, D), lambda i: (0, 0)),
                pl.BlockSpec((E, 1), lambda i: (0, 0)),
            ],
            out_specs=pl.BlockSpec((E, TB), lambda i: (0, i)),
            out_shape=jax.ShapeDtypeStruct((E, length), jnp.float32),
        )(xf, W, b2)

    sc0 = functools.partial(
        pl.kernel,
        out_type=[
            jax.ShapeDtypeStruct((E, T), jnp.float32),
            jax.ShapeDtypeStruct((T * 2,), jnp.int32),
        ],
        mesh=mesh,
        compiler_params=pltpu.CompilerParams(needs_layout_passes=False),
        scratch_types=_SC_SCRATCH,
    )(functools.partial(_sc_route_body, *SEGS[0]))

    sc_refs = [
        functools.partial(
            pl.kernel,
            out_type=(),
            mesh=mesh,
            compiler_params=pltpu.CompilerParams(needs_layout_passes=False),
            scratch_types=_SC_SCRATCH,
        )(functools.partial(_sc_route_body, *SEGS[s]))
        for s in range(1, len(SEGS))
    ]

    lg0 = tc_logits(*SEGS[0])
    rw0, idx0 = sc0(lg0)
    rw_ref = jax.new_ref(rw0)
    idx_ref = jax.new_ref(idx0)
    for s in range(1, len(SEGS)):
        lg = tc_logits(*SEGS[s])
        sc_refs[s - 1](lg, rw_ref, idx_ref)
    rw_t = rw_ref[...]
    idx_flat = idx_ref[...]
    rw = rw_t.T
    idx = idx_flat.reshape(T // 128, 2, 128).transpose(0, 2, 1).reshape(T, 2)
    return (rw, idx)
